# baseline (speedup 1.0000x reference)
import dataclasses
import functools

import jax
import jax.numpy as jnp
from jax import lax
from jax.experimental import pallas as pl
from jax.experimental.pallas import tpu as pltpu
from jax.experimental.pallas import tpu_sc as plsc

C_DIM = 256
OUT_DIM = 16
CW, CH, CD = 16, 16, 16
NCELL = CW * CH * CD
ACC_LEN = OUT_DIM * NCELL

NEG = -3.0e38


BLK = 2048


def _mlp_body(x_ref, w1t_ref, b1_ref, w2t_ref, b2_ref, f_ref, a_ref):
    x = x_ref[...]
    w1t = w1t_ref[...]
    h = (x[:, 0:1] * w1t[0:1, :]
         + x[:, 1:2] * w1t[1:2, :]
         + x[:, 2:3] * w1t[2:3, :])
    h = jnp.maximum(h + b1_ref[...], 0.0)
    f = lax.dot_general(h.astype(jnp.bfloat16), w2t_ref[...],
                        (((1,), (0,)), ((), ())),
                        preferred_element_type=jnp.float32)
    f_ref[...] = f + b2_ref[...]
    cell = jnp.clip(jnp.floor(x * 16.0).astype(jnp.int32), 0, 15)
    flat = cell[:, 0:1] * (CH * CD) + cell[:, 1:2] * CD + cell[:, 2:3]
    a_ref[...] = flat + lax.broadcasted_iota(jnp.int32, (BLK, OUT_DIM), 1) * NCELL


def _run_mlp(xf, W1, b1, W2, b2):
    m = xf.shape[0]
    grid = (m // BLK,)
    return pl.pallas_call(
        _mlp_body,
        grid=grid,
        in_specs=[
            pl.BlockSpec((BLK, 3), lambda i: (i, 0)),
            pl.BlockSpec((3, C_DIM), lambda i: (0, 0)),
            pl.BlockSpec((1, C_DIM), lambda i: (0, 0)),
            pl.BlockSpec((C_DIM, OUT_DIM), lambda i: (0, 0)),
            pl.BlockSpec((1, OUT_DIM), lambda i: (0, 0)),
        ],
        out_specs=[
            pl.BlockSpec((BLK, OUT_DIM), lambda i: (i, 0)),
            pl.BlockSpec((BLK, OUT_DIM), lambda i: (i, 0)),
        ],
        out_shape=[
            jax.ShapeDtypeStruct((m, OUT_DIM), jnp.float32),
            jax.ShapeDtypeStruct((m, OUT_DIM), jnp.int32),
        ],
    )(xf, W1.T, b1.reshape(1, C_DIM), W2.T.astype(jnp.bfloat16),
      b2.reshape(1, OUT_DIM))


NWORK = 32
CHUNK = 512
CH16 = CHUNK * OUT_DIM


def _sc_pool_body(f_hbm, a_hbm, out_hbm, fb0, fb1, ab0, ab1, acc,
                  sf0, sf1, sa0, sa1):
    c = lax.axis_index("c")
    s = lax.axis_index("s")
    wid = c * 16 + s
    npts = f_hbm.shape[0] // (NWORK * OUT_DIM)
    nch = npts // CHUNK
    base = wid * (npts * OUT_DIM)

    @pl.loop(0, ACC_LEN, step=16)
    def _(i):
        acc[pl.ds(i, 16)] = jnp.full((16,), NEG, jnp.float32)

    pltpu.async_copy(f_hbm.at[pl.ds(base, CH16)], fb0, sf0)
    pltpu.async_copy(a_hbm.at[pl.ds(base, CH16)], ab0, sa0)
    pltpu.async_copy(f_hbm.at[pl.ds(base + CH16, CH16)], fb1, sf1)
    pltpu.async_copy(a_hbm.at[pl.ds(base + CH16, CH16)], ab1, sa1)

    def proc(fb, ab):
        @pl.loop(0, CHUNK)
        def _(p):
            idx = ab[pl.ds(p * 16, 16)]
            val = fb[pl.ds(p * 16, 16)]
            old = plsc.load_gather(acc, [idx])
            plsc.store_scatter(acc, [idx], jnp.maximum(old, val))

    @pl.loop(0, nch, step=2)
    def _(ci):
        off0 = base + ci * CH16
        pltpu.make_async_copy(f_hbm.at[pl.ds(off0, CH16)], fb0, sf0).wait()
        pltpu.make_async_copy(a_hbm.at[pl.ds(off0, CH16)], ab0, sa0).wait()
        proc(fb0, ab0)

        @pl.when(ci + 2 < nch)
        def _():
            o2 = base + (ci + 2) * CH16
            pltpu.async_copy(f_hbm.at[pl.ds(o2, CH16)], fb0, sf0)
            pltpu.async_copy(a_hbm.at[pl.ds(o2, CH16)], ab0, sa0)

        off1 = base + (ci + 1) * CH16
        pltpu.make_async_copy(f_hbm.at[pl.ds(off1, CH16)], fb1, sf1).wait()
        pltpu.make_async_copy(a_hbm.at[pl.ds(off1, CH16)], ab1, sa1).wait()
        proc(fb1, ab1)

        @pl.when(ci + 3 < nch)
        def _():
            o3 = base + (ci + 3) * CH16
            pltpu.async_copy(f_hbm.at[pl.ds(o3, CH16)], fb1, sf1)
            pltpu.async_copy(a_hbm.at[pl.ds(o3, CH16)], ab1, sa1)

    pltpu.sync_copy(acc, out_hbm.at[wid])


def _run_sc_pool(f_flat, a_flat):
    mesh = plsc.VectorSubcoreMesh(core_axis_name="c", subcore_axis_name="s")
    cp = pltpu.CompilerParams()
    if "needs_layout_passes" in pltpu.CompilerParams.__dataclass_fields__:
        cp = dataclasses.replace(cp, needs_layout_passes=False)
    fn = pl.kernel(
        _sc_pool_body,
        out_type=jax.ShapeDtypeStruct((NWORK, ACC_LEN), jnp.float32),
        mesh=mesh,
        scratch_types=[
            pltpu.VMEM((CH16,), jnp.float32),
            pltpu.VMEM((CH16,), jnp.float32),
            pltpu.VMEM((CH16,), jnp.int32),
            pltpu.VMEM((CH16,), jnp.int32),
            pltpu.VMEM((ACC_LEN,), jnp.float32),
            pltpu.SemaphoreType.DMA,
            pltpu.SemaphoreType.DMA,
            pltpu.SemaphoreType.DMA,
            pltpu.SemaphoreType.DMA,
        ],
        compiler_params=cp,
    )
    return fn(f_flat, a_flat)


def _merge_body(p_ref, o_ref):
    m = jnp.maximum(p_ref[0, 0], p_ref[0, 1])
    o_ref[0] = jnp.where(m > -1.0e38, m, 0.0)


def _run_merge(partials, nb):
    p4 = partials.reshape(nb, 2, 512, 128)
    return pl.pallas_call(
        _merge_body,
        grid=(nb,),
        in_specs=[pl.BlockSpec((1, 2, 512, 128), lambda i: (i, 0, 0, 0))],
        out_specs=pl.BlockSpec((1, 512, 128), lambda i: (i, 0, 0)),
        out_shape=jax.ShapeDtypeStruct((nb, 512, 128), jnp.float32),
    )(p4)


def kernel(x, W1, b1, W2, b2):
    nb, npts, _ = x.shape
    xf = x.reshape(nb * npts, 3)
    f, a = _run_mlp(xf, W1, b1, W2, b2)
    partials = _run_sc_pool(f.reshape(-1), a.reshape(-1))
    merged = _run_merge(partials, nb)
    grid3 = merged.reshape(nb, OUT_DIM, CW, CH, CD)
    return jnp.pad(grid3, ((0, 0), (0, 0), (0, 1), (0, 1), (0, 1)))


# speedup vs baseline: 1.0798x; 1.0798x over previous
import dataclasses

import jax
import jax.numpy as jnp
from jax import lax
from jax.experimental import pallas as pl
from jax.experimental.pallas import tpu as pltpu
from jax.experimental.pallas import tpu_sc as plsc

C_DIM = 256
OUT_DIM = 16
CW, CH, CD = 16, 16, 16
NCELL = CW * CH * CD
ACC_LEN = OUT_DIM * NCELL

NEG = -3.0e38


BLK = 2048


def _mlp_body(x_ref, w1t_ref, b1_ref, w2t_ref, b2_ref, o_ref):
    x = x_ref[...]
    w1t = w1t_ref[...]
    h = (x[:, 0:1] * w1t[0:1, :]
         + x[:, 1:2] * w1t[1:2, :]
         + x[:, 2:3] * w1t[2:3, :])
    h = jnp.maximum(h + b1_ref[...], 0.0)
    f = lax.dot_general(h.astype(jnp.bfloat16), w2t_ref[...],
                        (((1,), (0,)), ((), ())),
                        preferred_element_type=jnp.float32)
    f = f + b2_ref[...]
    cell = jnp.clip(jnp.floor(x * 16.0).astype(jnp.int32), 0, 15)
    flat = cell[:, 0:1] * (CH * CD) + cell[:, 1:2] * CD + cell[:, 2:3]
    a = flat + lax.broadcasted_iota(jnp.int32, (BLK, OUT_DIM), 1) * NCELL
    af = lax.bitcast_convert_type(a, jnp.float32)
    sub = BLK // 4
    for j in range(4):
        o_ref[:, j * 32:j * 32 + OUT_DIM] = f[j * sub:(j + 1) * sub, :]
        o_ref[:, j * 32 + OUT_DIM:(j + 1) * 32] = af[j * sub:(j + 1) * sub, :]


def _run_mlp(xf, W1, b1, W2, b2):
    m = xf.shape[0]
    grid = (m // BLK,)
    return pl.pallas_call(
        _mlp_body,
        grid=grid,
        in_specs=[
            pl.BlockSpec((BLK, 3), lambda i: (i, 0)),
            pl.BlockSpec((3, C_DIM), lambda i: (0, 0)),
            pl.BlockSpec((1, C_DIM), lambda i: (0, 0)),
            pl.BlockSpec((C_DIM, OUT_DIM), lambda i: (0, 0)),
            pl.BlockSpec((1, OUT_DIM), lambda i: (0, 0)),
        ],
        out_specs=pl.BlockSpec((BLK // 4, 128), lambda i: (i, 0)),
        out_shape=jax.ShapeDtypeStruct((m // 4, 128), jnp.float32),
    )(xf, W1.T, b1.reshape(1, C_DIM), W2.T.astype(jnp.bfloat16),
      b2.reshape(1, OUT_DIM))


NWORK = 32
CHUNK = 512


CHW = CHUNK * 2 * OUT_DIM


def _sc_pool_body(fa_hbm, out_hbm, b0, b1, acc, s0, s1):
    c = lax.axis_index("c")
    s = lax.axis_index("s")
    wid = c * 16 + s
    npts = fa_hbm.shape[0] // (NWORK * 2 * OUT_DIM)
    nch = npts // CHUNK
    base = wid * npts * 2 * OUT_DIM

    @pl.loop(0, ACC_LEN, step=16)
    def _(i):
        acc[pl.ds(i, 16)] = jnp.full((16,), NEG, jnp.float32)

    pltpu.async_copy(fa_hbm.at[pl.ds(base, CHW)], b0, s0)
    pltpu.async_copy(fa_hbm.at[pl.ds(base + CHW, CHW)], b1, s1)

    def proc(buf):
        @pl.loop(0, CHUNK)
        def _(p):
            val = buf[pl.ds(p * 32, OUT_DIM)]
            idx = plsc.bitcast(buf[pl.ds(p * 32 + OUT_DIM, OUT_DIM)],
                               jnp.int32)
            old = plsc.load_gather(acc, [idx])
            plsc.store_scatter(acc, [idx], jnp.maximum(old, val))

    @pl.loop(0, nch, step=2)
    def _(ci):
        r0 = base + ci * CHW
        pltpu.make_async_copy(fa_hbm.at[pl.ds(r0, CHW)], b0, s0).wait()
        proc(b0)

        @pl.when(ci + 2 < nch)
        def _():
            pltpu.async_copy(
                fa_hbm.at[pl.ds(base + (ci + 2) * CHW, CHW)], b0, s0)

        pltpu.make_async_copy(
            fa_hbm.at[pl.ds(base + (ci + 1) * CHW, CHW)], b1, s1).wait()
        proc(b1)

        @pl.when(ci + 3 < nch)
        def _():
            pltpu.async_copy(
                fa_hbm.at[pl.ds(base + (ci + 3) * CHW, CHW)], b1, s1)

    pltpu.sync_copy(acc, out_hbm.at[wid])


def _run_sc_pool(fa):
    mesh = plsc.VectorSubcoreMesh(core_axis_name="c", subcore_axis_name="s")
    cp = pltpu.CompilerParams()
    if "needs_layout_passes" in pltpu.CompilerParams.__dataclass_fields__:
        cp = dataclasses.replace(cp, needs_layout_passes=False)
    fn = pl.kernel(
        _sc_pool_body,
        out_type=jax.ShapeDtypeStruct((NWORK, ACC_LEN), jnp.float32),
        mesh=mesh,
        scratch_types=[
            pltpu.VMEM((CHW,), jnp.float32),
            pltpu.VMEM((CHW,), jnp.float32),
            pltpu.VMEM((ACC_LEN,), jnp.float32),
            pltpu.SemaphoreType.DMA,
            pltpu.SemaphoreType.DMA,
        ],
        compiler_params=cp,
    )
    return fn(fa)


def _merge_body(p_ref, o_ref):
    m = jnp.maximum(p_ref[0, 0], p_ref[0, 1])
    o_ref[0] = jnp.where(m > -1.0e38, m, 0.0)


def _run_merge(partials, nb):
    p4 = partials.reshape(nb, 2, 512, 128)
    return pl.pallas_call(
        _merge_body,
        grid=(nb,),
        in_specs=[pl.BlockSpec((1, 2, 512, 128), lambda i: (i, 0, 0, 0))],
        out_specs=pl.BlockSpec((1, 512, 128), lambda i: (i, 0, 0)),
        out_shape=jax.ShapeDtypeStruct((nb, 512, 128), jnp.float32),
    )(p4)


def kernel(x, W1, b1, W2, b2):
    nb, npts, _ = x.shape
    xf = x.reshape(nb * npts, 3)
    fa = _run_mlp(xf, W1, b1, W2, b2)
    partials = _run_sc_pool(fa.reshape(-1))
    merged = _run_merge(partials, nb)
    grid3 = merged.reshape(nb, OUT_DIM, CW, CH, CD)
    return jnp.pad(grid3, ((0, 0), (0, 0), (0, 1), (0, 1), (0, 1)))


# speedup vs baseline: 3.4365x; 3.1826x over previous
import dataclasses

import jax
import jax.numpy as jnp
from jax import lax
from jax.experimental import pallas as pl
from jax.experimental.pallas import tpu as pltpu
from jax.experimental.pallas import tpu_sc as plsc

C_DIM = 256
OUT_DIM = 16
CW, CH, CD = 16, 16, 16
NCELL = CW * CH * CD
ACC_LEN = OUT_DIM * NCELL

NEG = -3.0e38


BLK = 2048


def _mlp_body(x_ref, w1t_ref, b1_ref, w2t_ref, b2_ref, o_ref):
    xt = x_ref[...]
    h = lax.dot_general(xt, w1t_ref[...], (((0,), (0,)), ((), ())),
                        preferred_element_type=jnp.float32)
    h = jnp.maximum(h + b1_ref[...], 0.0)
    f = lax.dot_general(h.astype(jnp.bfloat16), w2t_ref[...],
                        (((1,), (0,)), ((), ())),
                        preferred_element_type=jnp.float32)
    f = f + b2_ref[...]
    cell = jnp.clip(jnp.floor(xt * 16.0).astype(jnp.int32), 0, 15)
    hi = (cell[0:1, :] * CH + cell[1:2, :]).astype(jnp.bfloat16)
    lo = cell[2:3, :].astype(jnp.bfloat16)
    rs = jnp.concatenate(
        [jnp.full((1, OUT_DIM), CD, jnp.bfloat16),
         jnp.full((1, OUT_DIM), 1, jnp.bfloat16)], axis=0)
    rep = lax.dot_general(jnp.concatenate([hi, lo], axis=0), rs,
                          (((0,), (0,)), ((), ())),
                          preferred_element_type=jnp.float32)
    a = rep.astype(jnp.int32) + \
        lax.broadcasted_iota(jnp.int32, (BLK, OUT_DIM), 1) * NCELL
    af = lax.bitcast_convert_type(a, jnp.float32)
    sub = BLK // 4
    for j in range(4):
        o_ref[:, j * 32:j * 32 + OUT_DIM] = f[j * sub:(j + 1) * sub, :]
        o_ref[:, j * 32 + OUT_DIM:(j + 1) * 32] = af[j * sub:(j + 1) * sub, :]


def _run_mlp(xt, W1, b1, W2, b2):
    m = xt.shape[1]
    grid = (m // BLK,)
    return pl.pallas_call(
        _mlp_body,
        grid=grid,
        in_specs=[
            pl.BlockSpec((3, BLK), lambda i: (0, i)),
            pl.BlockSpec((3, C_DIM), lambda i: (0, 0)),
            pl.BlockSpec((1, C_DIM), lambda i: (0, 0)),
            pl.BlockSpec((C_DIM, OUT_DIM), lambda i: (0, 0)),
            pl.BlockSpec((1, OUT_DIM), lambda i: (0, 0)),
        ],
        out_specs=pl.BlockSpec((BLK // 4, 128), lambda i: (i, 0)),
        out_shape=jax.ShapeDtypeStruct((m // 4, 128), jnp.float32),
    )(xt, W1.T, b1.reshape(1, C_DIM), W2.T.astype(jnp.bfloat16),
      b2.reshape(1, OUT_DIM))


NWORK = 32
CHUNK = 512


CHW = CHUNK * 2 * OUT_DIM


def _sc_pool_body(fa_hbm, out_hbm, b0, b1, acc, s0, s1):
    c = lax.axis_index("c")
    s = lax.axis_index("s")
    wid = c * 16 + s
    npts = fa_hbm.shape[0] // (NWORK * 2 * OUT_DIM)
    nch = npts // CHUNK
    base = wid * npts * 2 * OUT_DIM

    @pl.loop(0, ACC_LEN, step=16)
    def _(i):
        acc[pl.ds(i, 16)] = jnp.full((16,), NEG, jnp.float32)

    pltpu.async_copy(fa_hbm.at[pl.ds(base, CHW)], b0, s0)
    pltpu.async_copy(fa_hbm.at[pl.ds(base + CHW, CHW)], b1, s1)

    def proc(buf):
        @pl.loop(0, CHUNK)
        def _(p):
            val = buf[pl.ds(p * 32, OUT_DIM)]
            idx = plsc.bitcast(buf[pl.ds(p * 32 + OUT_DIM, OUT_DIM)],
                               jnp.int32)
            old = plsc.load_gather(acc, [idx])
            plsc.store_scatter(acc, [idx], jnp.maximum(old, val))

    @pl.loop(0, nch, step=2)
    def _(ci):
        r0 = base + ci * CHW
        pltpu.make_async_copy(fa_hbm.at[pl.ds(r0, CHW)], b0, s0).wait()
        proc(b0)

        @pl.when(ci + 2 < nch)
        def _():
            pltpu.async_copy(
                fa_hbm.at[pl.ds(base + (ci + 2) * CHW, CHW)], b0, s0)

        pltpu.make_async_copy(
            fa_hbm.at[pl.ds(base + (ci + 1) * CHW, CHW)], b1, s1).wait()
        proc(b1)

        @pl.when(ci + 3 < nch)
        def _():
            pltpu.async_copy(
                fa_hbm.at[pl.ds(base + (ci + 3) * CHW, CHW)], b1, s1)

    pltpu.sync_copy(acc, out_hbm.at[wid])


def _run_sc_pool(fa):
    mesh = plsc.VectorSubcoreMesh(core_axis_name="c", subcore_axis_name="s")
    cp = pltpu.CompilerParams()
    if "needs_layout_passes" in pltpu.CompilerParams.__dataclass_fields__:
        cp = dataclasses.replace(cp, needs_layout_passes=False)
    fn = pl.kernel(
        _sc_pool_body,
        out_type=jax.ShapeDtypeStruct((NWORK, ACC_LEN), jnp.float32),
        mesh=mesh,
        scratch_types=[
            pltpu.VMEM((CHW,), jnp.float32),
            pltpu.VMEM((CHW,), jnp.float32),
            pltpu.VMEM((ACC_LEN,), jnp.float32),
            pltpu.SemaphoreType.DMA,
            pltpu.SemaphoreType.DMA,
        ],
        compiler_params=cp,
    )
    return fn(fa)


def _merge_body(p_ref, o_ref):
    m = jnp.maximum(p_ref[0, 0], p_ref[0, 1])
    o_ref[0] = jnp.where(m > -1.0e38, m, 0.0)


def _run_merge(partials, nb):
    p4 = partials.reshape(nb, 2, 512, 128)
    return pl.pallas_call(
        _merge_body,
        grid=(nb,),
        in_specs=[pl.BlockSpec((1, 2, 512, 128), lambda i: (i, 0, 0, 0))],
        out_specs=pl.BlockSpec((1, 512, 128), lambda i: (i, 0, 0)),
        out_shape=jax.ShapeDtypeStruct((nb, 512, 128), jnp.float32),
    )(p4)


def kernel(x, W1, b1, W2, b2):
    nb, npts, _ = x.shape
    xt = x.transpose(2, 0, 1).reshape(3, nb * npts)
    fa = _run_mlp(xt, W1, b1, W2, b2)
    partials = _run_sc_pool(fa.reshape(-1))
    merged = _run_merge(partials, nb)
    grid3 = merged.reshape(nb, OUT_DIM, CW, CH, CD)
    return jnp.pad(grid3, ((0, 0), (0, 0), (0, 1), (0, 1), (0, 1)))


# speedup vs baseline: 4.6155x; 1.3431x over previous
import dataclasses

import jax
import jax.numpy as jnp
from jax import lax
from jax.experimental import pallas as pl
from jax.experimental.pallas import tpu as pltpu
from jax.experimental.pallas import tpu_sc as plsc

C_DIM = 256
OUT_DIM = 16
CW, CH, CD = 16, 16, 16
NCELL = CW * CH * CD
PITCH = NCELL + 17
ACC_LEN = OUT_DIM * PITCH

NEG = -3.0e38


BLK = 2048


def _mlp_body(x_ref, w1t_ref, b1_ref, w2t_ref, b2_ref, o_ref):
    xt = x_ref[...]
    h = lax.dot_general(xt, w1t_ref[...], (((0,), (0,)), ((), ())),
                        preferred_element_type=jnp.float32)
    h = jnp.maximum(h + b1_ref[...], 0.0)
    f = lax.dot_general(h.astype(jnp.bfloat16), w2t_ref[...],
                        (((1,), (0,)), ((), ())),
                        preferred_element_type=jnp.float32)
    f = f + b2_ref[...]
    cell = jnp.clip(jnp.floor(xt * 16.0).astype(jnp.int32), 0, 15)
    hi = (cell[0:1, :] * CH + cell[1:2, :]).astype(jnp.bfloat16)
    lo = cell[2:3, :].astype(jnp.bfloat16)
    rs = jnp.concatenate(
        [jnp.full((1, OUT_DIM), CD, jnp.bfloat16),
         jnp.full((1, OUT_DIM), 1, jnp.bfloat16)], axis=0)
    rep = lax.dot_general(jnp.concatenate([hi, lo], axis=0), rs,
                          (((0,), (0,)), ((), ())),
                          preferred_element_type=jnp.float32)
    a = rep.astype(jnp.int32) + \
        lax.broadcasted_iota(jnp.int32, (BLK, OUT_DIM), 1) * PITCH
    af = lax.bitcast_convert_type(a, jnp.float32)
    sub = BLK // 4
    for j in range(4):
        o_ref[:, j * 32:j * 32 + OUT_DIM] = f[j * sub:(j + 1) * sub, :]
        o_ref[:, j * 32 + OUT_DIM:(j + 1) * 32] = af[j * sub:(j + 1) * sub, :]


def _run_mlp(xt, W1, b1, W2, b2):
    m = xt.shape[1]
    grid = (m // BLK,)
    return pl.pallas_call(
        _mlp_body,
        grid=grid,
        in_specs=[
            pl.BlockSpec((3, BLK), lambda i: (0, i)),
            pl.BlockSpec((3, C_DIM), lambda i: (0, 0)),
            pl.BlockSpec((1, C_DIM), lambda i: (0, 0)),
            pl.BlockSpec((C_DIM, OUT_DIM), lambda i: (0, 0)),
            pl.BlockSpec((1, OUT_DIM), lambda i: (0, 0)),
        ],
        out_specs=pl.BlockSpec((BLK // 4, 128), lambda i: (i, 0)),
        out_shape=jax.ShapeDtypeStruct((m // 4, 128), jnp.float32),
    )(xt, W1.T, b1.reshape(1, C_DIM), W2.T.astype(jnp.bfloat16),
      b2.reshape(1, OUT_DIM))


NWORK = 32
CHUNK = 512


CHW = CHUNK * 2 * OUT_DIM


def _sc_pool_body(fa_hbm, out_hbm, b0, b1, acc, s0, s1):
    c = lax.axis_index("c")
    s = lax.axis_index("s")
    wid = c * 16 + s
    npts = fa_hbm.shape[0] // (NWORK * 2 * OUT_DIM)
    nch = npts // CHUNK
    base = wid * npts * 2 * OUT_DIM

    @pl.loop(0, ACC_LEN, step=16)
    def _(i):
        acc[pl.ds(i, 16)] = jnp.full((16,), NEG, jnp.float32)

    pltpu.async_copy(fa_hbm.at[pl.ds(base, CHW)], b0, s0)
    pltpu.async_copy(fa_hbm.at[pl.ds(base + CHW, CHW)], b1, s1)

    def proc(buf):
        @pl.loop(0, CHUNK)
        def _(p):
            val = buf[pl.ds(p * 32, OUT_DIM)]
            idx = plsc.bitcast(buf[pl.ds(p * 32 + OUT_DIM, OUT_DIM)],
                               jnp.int32)
            old = plsc.load_gather(acc, [idx])
            plsc.store_scatter(acc, [idx], jnp.maximum(old, val))

    @pl.loop(0, nch, step=2)
    def _(ci):
        r0 = base + ci * CHW
        pltpu.make_async_copy(fa_hbm.at[pl.ds(r0, CHW)], b0, s0).wait()
        proc(b0)

        @pl.when(ci + 2 < nch)
        def _():
            pltpu.async_copy(
                fa_hbm.at[pl.ds(base + (ci + 2) * CHW, CHW)], b0, s0)

        pltpu.make_async_copy(
            fa_hbm.at[pl.ds(base + (ci + 1) * CHW, CHW)], b1, s1).wait()
        proc(b1)

        @pl.when(ci + 3 < nch)
        def _():
            pltpu.async_copy(
                fa_hbm.at[pl.ds(base + (ci + 3) * CHW, CHW)], b1, s1)

    lane = lax.broadcasted_iota(jnp.int32, (16,), 0)

    @pl.loop(0, OUT_DIM)
    def _(f):
        @pl.loop(0, NCELL, step=16)
        def _(cc):
            b0[pl.ds(cc, 16)] = plsc.load_gather(
                acc, [f * PITCH + cc + lane])
        pltpu.sync_copy(b0.at[pl.ds(0, NCELL)],
                        out_hbm.at[wid, pl.ds(f * NCELL, NCELL)])


def _run_sc_pool(fa):
    mesh = plsc.VectorSubcoreMesh(core_axis_name="c", subcore_axis_name="s")
    cp = pltpu.CompilerParams()
    if "needs_layout_passes" in pltpu.CompilerParams.__dataclass_fields__:
        cp = dataclasses.replace(cp, needs_layout_passes=False)
    fn = pl.kernel(
        _sc_pool_body,
        out_type=jax.ShapeDtypeStruct((NWORK, OUT_DIM * NCELL), jnp.float32),
        mesh=mesh,
        scratch_types=[
            pltpu.VMEM((CHW,), jnp.float32),
            pltpu.VMEM((CHW,), jnp.float32),
            pltpu.VMEM((ACC_LEN,), jnp.float32),
            pltpu.SemaphoreType.DMA,
            pltpu.SemaphoreType.DMA,
        ],
        compiler_params=cp,
    )
    return fn(fa)


def _merge_body(p_ref, o_ref):
    m = jnp.maximum(p_ref[0, 0], p_ref[0, 1])
    o_ref[0] = jnp.where(m > -1.0e38, m, 0.0)


def _run_merge(partials, nb):
    p4 = partials.reshape(nb, 2, 512, 128)
    return pl.pallas_call(
        _merge_body,
        grid=(nb,),
        in_specs=[pl.BlockSpec((1, 2, 512, 128), lambda i: (i, 0, 0, 0))],
        out_specs=pl.BlockSpec((1, 512, 128), lambda i: (i, 0, 0)),
        out_shape=jax.ShapeDtypeStruct((nb, 512, 128), jnp.float32),
    )(p4)


def kernel(x, W1, b1, W2, b2):
    nb, npts, _ = x.shape
    xt = x.transpose(2, 0, 1).reshape(3, nb * npts)
    fa = _run_mlp(xt, W1, b1, W2, b2)
    partials = _run_sc_pool(fa.reshape(-1))
    merged = _run_merge(partials, nb)
    grid3 = merged.reshape(nb, OUT_DIM, CW, CH, CD)
    return jnp.pad(grid3, ((0, 0), (0, 0), (0, 1), (0, 1), (0, 1)))


# speedup vs baseline: 4.7177x; 1.0221x over previous
import dataclasses

import jax
import jax.numpy as jnp
from jax import lax
from jax.experimental import pallas as pl
from jax.experimental.pallas import tpu as pltpu
from jax.experimental.pallas import tpu_sc as plsc

C_DIM = 256
OUT_DIM = 16
CW, CH, CD = 16, 16, 16
NCELL = CW * CH * CD
PITCH = NCELL + 17
ACC_LEN = OUT_DIM * PITCH

NEG = -3.0e38


BLK = 2048


def _mlp_body(x_ref, w1t_ref, b1_ref, w2t_ref, b2_ref, o_ref):
    xt = x_ref[...]
    h = lax.dot_general(xt.astype(jnp.bfloat16), w1t_ref[...],
                        (((0,), (0,)), ((), ())),
                        preferred_element_type=jnp.float32)
    h = jnp.maximum(h + b1_ref[...], 0.0)
    f = lax.dot_general(h.astype(jnp.bfloat16), w2t_ref[...],
                        (((1,), (0,)), ((), ())),
                        preferred_element_type=jnp.float32)
    f = f + b2_ref[...]
    cell = jnp.clip(jnp.floor(xt * 16.0).astype(jnp.int32), 0, 15)
    hi = (cell[0:1, :] * CH + cell[1:2, :]).astype(jnp.bfloat16)
    lo = cell[2:3, :].astype(jnp.bfloat16)
    rs = jnp.concatenate(
        [jnp.full((1, OUT_DIM), CD, jnp.bfloat16),
         jnp.full((1, OUT_DIM), 1, jnp.bfloat16)], axis=0)
    rep = lax.dot_general(jnp.concatenate([hi, lo], axis=0), rs,
                          (((0,), (0,)), ((), ())),
                          preferred_element_type=jnp.float32)
    a = rep.astype(jnp.int32) + \
        lax.broadcasted_iota(jnp.int32, (BLK, OUT_DIM), 1) * PITCH
    af = lax.bitcast_convert_type(a, jnp.float32)
    sub = BLK // 4
    for j in range(4):
        o_ref[:, j * 32:j * 32 + OUT_DIM] = f[j * sub:(j + 1) * sub, :]
        o_ref[:, j * 32 + OUT_DIM:(j + 1) * 32] = af[j * sub:(j + 1) * sub, :]


def _run_mlp(xt, W1, b1, W2, b2):
    m = xt.shape[1]
    grid = (m // BLK,)
    return pl.pallas_call(
        _mlp_body,
        grid=grid,
        in_specs=[
            pl.BlockSpec((3, BLK), lambda i: (0, i)),
            pl.BlockSpec((3, C_DIM), lambda i: (0, 0)),
            pl.BlockSpec((1, C_DIM), lambda i: (0, 0)),
            pl.BlockSpec((C_DIM, OUT_DIM), lambda i: (0, 0)),
            pl.BlockSpec((1, OUT_DIM), lambda i: (0, 0)),
        ],
        out_specs=pl.BlockSpec((BLK // 4, 128), lambda i: (i, 0)),
        out_shape=jax.ShapeDtypeStruct((m // 4, 128), jnp.float32),
    )(xt, W1.T.astype(jnp.bfloat16), b1.reshape(1, C_DIM),
      W2.T.astype(jnp.bfloat16), b2.reshape(1, OUT_DIM))


NWORK = 32
CHUNK = 512


CHW = CHUNK * 2 * OUT_DIM


def _sc_pool_body(fa_hbm, out_hbm, b0, b1, acc, s0, s1):
    c = lax.axis_index("c")
    s = lax.axis_index("s")
    wid = c * 16 + s
    npts = fa_hbm.shape[0] // (NWORK * 2 * OUT_DIM)
    nch = npts // CHUNK
    base = wid * npts * 2 * OUT_DIM

    @pl.loop(0, ACC_LEN, step=16)
    def _(i):
        acc[pl.ds(i, 16)] = jnp.full((16,), NEG, jnp.float32)

    pltpu.async_copy(fa_hbm.at[pl.ds(base, CHW)], b0, s0)
    pltpu.async_copy(fa_hbm.at[pl.ds(base + CHW, CHW)], b1, s1)

    def proc(buf):
        @pl.loop(0, CHUNK)
        def _(p):
            val = buf[pl.ds(p * 32, OUT_DIM)]
            idx = plsc.bitcast(buf[pl.ds(p * 32 + OUT_DIM, OUT_DIM)],
                               jnp.int32)
            old = plsc.load_gather(acc, [idx])
            plsc.store_scatter(acc, [idx], jnp.maximum(old, val))

    @pl.loop(0, nch, step=2)
    def _(ci):
        r0 = base + ci * CHW
        pltpu.make_async_copy(fa_hbm.at[pl.ds(r0, CHW)], b0, s0).wait()
        proc(b0)

        @pl.when(ci + 2 < nch)
        def _():
            pltpu.async_copy(
                fa_hbm.at[pl.ds(base + (ci + 2) * CHW, CHW)], b0, s0)

        pltpu.make_async_copy(
            fa_hbm.at[pl.ds(base + (ci + 1) * CHW, CHW)], b1, s1).wait()
        proc(b1)

        @pl.when(ci + 3 < nch)
        def _():
            pltpu.async_copy(
                fa_hbm.at[pl.ds(base + (ci + 3) * CHW, CHW)], b1, s1)

    lane = lax.broadcasted_iota(jnp.int32, (16,), 0)

    @pl.loop(0, OUT_DIM)
    def _(f):
        @pl.loop(0, NCELL, step=16)
        def _(cc):
            b0[pl.ds(cc, 16)] = plsc.load_gather(
                acc, [f * PITCH + cc + lane])
        pltpu.sync_copy(b0.at[pl.ds(0, NCELL)],
                        out_hbm.at[wid, pl.ds(f * NCELL, NCELL)])


def _run_sc_pool(fa):
    mesh = plsc.VectorSubcoreMesh(core_axis_name="c", subcore_axis_name="s")
    cp = pltpu.CompilerParams()
    if "needs_layout_passes" in pltpu.CompilerParams.__dataclass_fields__:
        cp = dataclasses.replace(cp, needs_layout_passes=False)
    fn = pl.kernel(
        _sc_pool_body,
        out_type=jax.ShapeDtypeStruct((NWORK, OUT_DIM * NCELL), jnp.float32),
        mesh=mesh,
        scratch_types=[
            pltpu.VMEM((CHW,), jnp.float32),
            pltpu.VMEM((CHW,), jnp.float32),
            pltpu.VMEM((ACC_LEN,), jnp.float32),
            pltpu.SemaphoreType.DMA,
            pltpu.SemaphoreType.DMA,
        ],
        compiler_params=cp,
    )
    return fn(fa)


def _merge_body(p_ref, o_ref):
    m = jnp.maximum(p_ref[0, 0], p_ref[0, 1])
    o_ref[0] = jnp.where(m > -1.0e38, m, 0.0)


def _run_merge(partials, nb):
    p4 = partials.reshape(nb, 2, 512, 128)
    return pl.pallas_call(
        _merge_body,
        grid=(nb,),
        in_specs=[pl.BlockSpec((1, 2, 512, 128), lambda i: (i, 0, 0, 0))],
        out_specs=pl.BlockSpec((1, 512, 128), lambda i: (i, 0, 0)),
        out_shape=jax.ShapeDtypeStruct((nb, 512, 128), jnp.float32),
    )(p4)


def kernel(x, W1, b1, W2, b2):
    nb, npts, _ = x.shape
    xt = x.transpose(2, 0, 1).reshape(3, nb * npts)
    fa = _run_mlp(xt, W1, b1, W2, b2)
    partials = _run_sc_pool(fa.reshape(-1))
    merged = _run_merge(partials, nb)
    grid3 = merged.reshape(nb, OUT_DIM, CW, CH, CD)
    return jnp.pad(grid3, ((0, 0), (0, 0), (0, 1), (0, 1), (0, 1)))


# speedup vs baseline: 4.8264x; 1.0230x over previous
import dataclasses

import jax
import jax.numpy as jnp
from jax import lax
from jax.experimental import pallas as pl
from jax.experimental.pallas import tpu as pltpu
from jax.experimental.pallas import tpu_sc as plsc

C_DIM = 256
OUT_DIM = 16
CW, CH, CD = 16, 16, 16
NCELL = CW * CH * CD
PITCH = NCELL + 17
ACC_LEN = OUT_DIM * PITCH

NEG = -3.0e38


BLK = 2048


def _mlp_body(x_ref, w1t_ref, b1_ref, w2t_ref, b2_ref, o_ref):
    xt = x_ref[...]
    h = lax.dot_general(xt.astype(jnp.bfloat16), w1t_ref[...],
                        (((0,), (0,)), ((), ())),
                        preferred_element_type=jnp.float32)
    h = jnp.maximum(h + b1_ref[...], 0.0)
    f = lax.dot_general(h.astype(jnp.bfloat16), w2t_ref[...],
                        (((1,), (0,)), ((), ())),
                        preferred_element_type=jnp.float32)
    f = f + b2_ref[...]
    cell = jnp.clip(jnp.floor(xt * 16.0).astype(jnp.int32), 0, 15)
    hi = (cell[0:1, :] * CH + cell[1:2, :]).astype(jnp.bfloat16)
    lo = cell[2:3, :].astype(jnp.bfloat16)
    rs = jnp.concatenate(
        [jnp.full((1, OUT_DIM), CD, jnp.bfloat16),
         jnp.full((1, OUT_DIM), 1, jnp.bfloat16)], axis=0)
    rep = lax.dot_general(jnp.concatenate([hi, lo], axis=0), rs,
                          (((0,), (0,)), ((), ())),
                          preferred_element_type=jnp.float32)
    a = rep.astype(jnp.int32) + \
        lax.broadcasted_iota(jnp.int32, (BLK, OUT_DIM), 1) * PITCH
    af = lax.bitcast_convert_type(a, jnp.float32)
    sub = BLK // 4
    for j in range(4):
        o_ref[:, j * 32:j * 32 + OUT_DIM] = f[j * sub:(j + 1) * sub, :]
        o_ref[:, j * 32 + OUT_DIM:(j + 1) * 32] = af[j * sub:(j + 1) * sub, :]


def _run_mlp(xt, W1, b1, W2, b2):
    m = xt.shape[1]
    grid = (m // BLK,)
    return pl.pallas_call(
        _mlp_body,
        grid=grid,
        in_specs=[
            pl.BlockSpec((3, BLK), lambda i: (0, i)),
            pl.BlockSpec((3, C_DIM), lambda i: (0, 0)),
            pl.BlockSpec((1, C_DIM), lambda i: (0, 0)),
            pl.BlockSpec((C_DIM, OUT_DIM), lambda i: (0, 0)),
            pl.BlockSpec((1, OUT_DIM), lambda i: (0, 0)),
        ],
        out_specs=pl.BlockSpec((BLK // 4, 128), lambda i: (i, 0)),
        out_shape=jax.ShapeDtypeStruct((m // 4, 128), jnp.float32),
    )(xt, W1.T.astype(jnp.bfloat16), b1.reshape(1, C_DIM),
      W2.T.astype(jnp.bfloat16), b2.reshape(1, OUT_DIM))


NWORK = 32
CHUNK = 512


CHW = CHUNK * 2 * OUT_DIM


def _sc_pool_body(fa_hbm, out_hbm, b0, b1, acc, s0, s1):
    c = lax.axis_index("c")
    s = lax.axis_index("s")
    wid = c * 16 + s
    npts = fa_hbm.shape[0] // (NWORK * 2 * OUT_DIM)
    nch = npts // CHUNK
    base = wid * npts * 2 * OUT_DIM

    @pl.loop(0, ACC_LEN, step=16)
    def _(i):
        acc[pl.ds(i, 16)] = jnp.full((16,), NEG, jnp.float32)

    pltpu.async_copy(fa_hbm.at[pl.ds(base, CHW)], b0, s0)
    pltpu.async_copy(fa_hbm.at[pl.ds(base + CHW, CHW)], b1, s1)

    def proc(buf):
        @pl.loop(0, CHUNK, step=4)
        def _(p):
            for u in range(4):
                val = buf[pl.ds((p + u) * 32, OUT_DIM)]
                idx = plsc.bitcast(
                    buf[pl.ds((p + u) * 32 + OUT_DIM, OUT_DIM)], jnp.int32)
                old = plsc.load_gather(acc, [idx])
                plsc.store_scatter(acc, [idx], jnp.maximum(old, val))

    @pl.loop(0, nch, step=2)
    def _(ci):
        r0 = base + ci * CHW
        pltpu.make_async_copy(fa_hbm.at[pl.ds(r0, CHW)], b0, s0).wait()
        proc(b0)

        @pl.when(ci + 2 < nch)
        def _():
            pltpu.async_copy(
                fa_hbm.at[pl.ds(base + (ci + 2) * CHW, CHW)], b0, s0)

        pltpu.make_async_copy(
            fa_hbm.at[pl.ds(base + (ci + 1) * CHW, CHW)], b1, s1).wait()
        proc(b1)

        @pl.when(ci + 3 < nch)
        def _():
            pltpu.async_copy(
                fa_hbm.at[pl.ds(base + (ci + 3) * CHW, CHW)], b1, s1)

    lane = lax.broadcasted_iota(jnp.int32, (16,), 0)

    @pl.loop(0, OUT_DIM)
    def _(f):
        @pl.loop(0, NCELL, step=16)
        def _(cc):
            b0[pl.ds(cc, 16)] = plsc.load_gather(
                acc, [f * PITCH + cc + lane])
        pltpu.sync_copy(b0.at[pl.ds(0, NCELL)],
                        out_hbm.at[wid, pl.ds(f * NCELL, NCELL)])


def _run_sc_pool(fa):
    mesh = plsc.VectorSubcoreMesh(core_axis_name="c", subcore_axis_name="s")
    cp = pltpu.CompilerParams()
    if "needs_layout_passes" in pltpu.CompilerParams.__dataclass_fields__:
        cp = dataclasses.replace(cp, needs_layout_passes=False)
    fn = pl.kernel(
        _sc_pool_body,
        out_type=jax.ShapeDtypeStruct((NWORK, OUT_DIM * NCELL), jnp.float32),
        mesh=mesh,
        scratch_types=[
            pltpu.VMEM((CHW,), jnp.float32),
            pltpu.VMEM((CHW,), jnp.float32),
            pltpu.VMEM((ACC_LEN,), jnp.float32),
            pltpu.SemaphoreType.DMA,
            pltpu.SemaphoreType.DMA,
        ],
        compiler_params=cp,
    )
    return fn(fa)


def _merge_body(p_ref, o_ref):
    m = jnp.maximum(p_ref[0, 0], p_ref[0, 1])
    o_ref[0] = jnp.where(m > -1.0e38, m, 0.0)


def _run_merge(partials, nb):
    p4 = partials.reshape(nb, 2, 512, 128)
    return pl.pallas_call(
        _merge_body,
        grid=(nb,),
        in_specs=[pl.BlockSpec((1, 2, 512, 128), lambda i: (i, 0, 0, 0))],
        out_specs=pl.BlockSpec((1, 512, 128), lambda i: (i, 0, 0)),
        out_shape=jax.ShapeDtypeStruct((nb, 512, 128), jnp.float32),
    )(p4)


def kernel(x, W1, b1, W2, b2):
    nb, npts, _ = x.shape
    xt = x.transpose(2, 0, 1).reshape(3, nb * npts)
    fa = _run_mlp(xt, W1, b1, W2, b2)
    partials = _run_sc_pool(fa.reshape(-1))
    merged = _run_merge(partials, nb)
    grid3 = merged.reshape(nb, OUT_DIM, CW, CH, CD)
    return jnp.pad(grid3, ((0, 0), (0, 0), (0, 1), (0, 1), (0, 1)))


# speedup vs baseline: 5.0830x; 1.0532x over previous
import dataclasses

import jax
import jax.numpy as jnp
from jax import lax
from jax.experimental import pallas as pl
from jax.experimental.pallas import tpu as pltpu
from jax.experimental.pallas import tpu_sc as plsc

C_DIM = 256
OUT_DIM = 16
CW, CH, CD = 16, 16, 16
NCELL = CW * CH * CD
PITCH = NCELL + 17
ACC_LEN = OUT_DIM * PITCH

NEG = -3.0e38


BLK = 2048


def _mlp_body(x_ref, w1t_ref, b1_ref, w2p_ref, rsp_ref, cb_ref, o_ref):
    xt = x_ref[...]
    h = lax.dot_general(xt.astype(jnp.bfloat16), w1t_ref[...],
                        (((0,), (0,)), ((), ())),
                        preferred_element_type=jnp.float32)
    hb = jnp.maximum(h + b1_ref[...], 0.0).astype(jnp.bfloat16)
    cell = jnp.clip(jnp.floor(xt * 16.0).astype(jnp.int32), 0, 15)
    hi = (cell[0:1, :] * CH + cell[1:2, :]).astype(jnp.bfloat16)
    lo = cell[2:3, :].astype(jnp.bfloat16)
    hl = jnp.concatenate([hi, lo], axis=0)
    sub = BLK // 4
    tot = cb_ref[...]
    for g in range(4):
        tot = tot + lax.dot_general(
            hb[g * sub:(g + 1) * sub], w2p_ref[g * C_DIM:(g + 1) * C_DIM],
            (((1,), (0,)), ((), ())), preferred_element_type=jnp.float32)
        tot = tot + lax.dot_general(
            hl[:, g * sub:(g + 1) * sub], rsp_ref[g * 2:g * 2 + 2],
            (((0,), (0,)), ((), ())), preferred_element_type=jnp.float32)
    ai = lax.bitcast_convert_type(tot.astype(jnp.int32), jnp.float32)
    is_feat = (lax.broadcasted_iota(jnp.int32, (sub, 128), 1)
               // OUT_DIM) % 2 == 0
    o_ref[...] = jnp.where(is_feat, tot, ai)


def _run_mlp(xt, W1, b1, W2, b2):
    m = xt.shape[1]
    grid = (m // BLK,)
    w2t = W2.T.astype(jnp.bfloat16)
    w2p = jnp.concatenate(
        [jnp.pad(w2t, ((0, 0), (g * 32, 112 - g * 32))) for g in range(4)],
        axis=0)
    rs = jnp.concatenate(
        [jnp.full((1, OUT_DIM), CD, jnp.bfloat16),
         jnp.full((1, OUT_DIM), 1, jnp.bfloat16)], axis=0)
    rsp = jnp.concatenate(
        [jnp.pad(rs, ((0, 0), (g * 32 + 16, 96 - g * 32))) for g in range(4)],
        axis=0)
    cb = jnp.tile(
        jnp.concatenate([b2, jnp.arange(OUT_DIM, dtype=jnp.float32) * PITCH]),
        4).reshape(1, 128)
    return pl.pallas_call(
        _mlp_body,
        grid=grid,
        in_specs=[
            pl.BlockSpec((3, BLK), lambda i: (0, i)),
            pl.BlockSpec((3, C_DIM), lambda i: (0, 0)),
            pl.BlockSpec((1, C_DIM), lambda i: (0, 0)),
            pl.BlockSpec((4 * C_DIM, 128), lambda i: (0, 0)),
            pl.BlockSpec((8, 128), lambda i: (0, 0)),
            pl.BlockSpec((1, 128), lambda i: (0, 0)),
        ],
        out_specs=pl.BlockSpec((BLK // 4, 128), lambda i: (i, 0)),
        out_shape=jax.ShapeDtypeStruct((m // 4, 128), jnp.float32),
    )(xt, W1.T.astype(jnp.bfloat16), b1.reshape(1, C_DIM), w2p, rsp, cb)


NWORK = 32
CHUNK = 512


CHW = CHUNK * 2 * OUT_DIM


def _sc_pool_body(fa_hbm, out_hbm, b0, b1, acc, s0, s1):
    c = lax.axis_index("c")
    s = lax.axis_index("s")
    wid = c * 16 + s
    npts = fa_hbm.shape[0] // (NWORK * 2 * OUT_DIM)
    nch = npts // CHUNK
    base = wid * npts * 2 * OUT_DIM

    @pl.loop(0, ACC_LEN, step=16)
    def _(i):
        acc[pl.ds(i, 16)] = jnp.full((16,), NEG, jnp.float32)

    pltpu.async_copy(fa_hbm.at[pl.ds(base, CHW)], b0, s0)
    pltpu.async_copy(fa_hbm.at[pl.ds(base + CHW, CHW)], b1, s1)

    def proc(buf):
        @pl.loop(0, CHUNK, step=4)
        def _(p):
            for u in range(4):
                val = buf[pl.ds((p + u) * 32, OUT_DIM)]
                idx = plsc.bitcast(
                    buf[pl.ds((p + u) * 32 + OUT_DIM, OUT_DIM)], jnp.int32)
                old = plsc.load_gather(acc, [idx])
                plsc.store_scatter(acc, [idx], jnp.maximum(old, val))

    @pl.loop(0, nch, step=2)
    def _(ci):
        r0 = base + ci * CHW
        pltpu.make_async_copy(fa_hbm.at[pl.ds(r0, CHW)], b0, s0).wait()
        proc(b0)

        @pl.when(ci + 2 < nch)
        def _():
            pltpu.async_copy(
                fa_hbm.at[pl.ds(base + (ci + 2) * CHW, CHW)], b0, s0)

        pltpu.make_async_copy(
            fa_hbm.at[pl.ds(base + (ci + 1) * CHW, CHW)], b1, s1).wait()
        proc(b1)

        @pl.when(ci + 3 < nch)
        def _():
            pltpu.async_copy(
                fa_hbm.at[pl.ds(base + (ci + 3) * CHW, CHW)], b1, s1)

    lane = lax.broadcasted_iota(jnp.int32, (16,), 0)

    @pl.loop(0, OUT_DIM)
    def _(f):
        @pl.loop(0, NCELL, step=16)
        def _(cc):
            b0[pl.ds(cc, 16)] = plsc.load_gather(
                acc, [f * PITCH + cc + lane])
        pltpu.sync_copy(b0.at[pl.ds(0, NCELL)],
                        out_hbm.at[wid, pl.ds(f * NCELL, NCELL)])


def _run_sc_pool(fa):
    mesh = plsc.VectorSubcoreMesh(core_axis_name="c", subcore_axis_name="s")
    cp = pltpu.CompilerParams()
    if "needs_layout_passes" in pltpu.CompilerParams.__dataclass_fields__:
        cp = dataclasses.replace(cp, needs_layout_passes=False)
    fn = pl.kernel(
        _sc_pool_body,
        out_type=jax.ShapeDtypeStruct((NWORK, OUT_DIM * NCELL), jnp.float32),
        mesh=mesh,
        scratch_types=[
            pltpu.VMEM((CHW,), jnp.float32),
            pltpu.VMEM((CHW,), jnp.float32),
            pltpu.VMEM((ACC_LEN,), jnp.float32),
            pltpu.SemaphoreType.DMA,
            pltpu.SemaphoreType.DMA,
        ],
        compiler_params=cp,
    )
    return fn(fa)


def _merge_body(p_ref, o_ref):
    m = jnp.maximum(p_ref[0, 0], p_ref[0, 1])
    o_ref[0] = jnp.where(m > -1.0e38, m, 0.0)


def _run_merge(partials, nb):
    p4 = partials.reshape(nb, 2, 512, 128)
    return pl.pallas_call(
        _merge_body,
        grid=(nb,),
        in_specs=[pl.BlockSpec((1, 2, 512, 128), lambda i: (i, 0, 0, 0))],
        out_specs=pl.BlockSpec((1, 512, 128), lambda i: (i, 0, 0)),
        out_shape=jax.ShapeDtypeStruct((nb, 512, 128), jnp.float32),
    )(p4)


def kernel(x, W1, b1, W2, b2):
    nb, npts, _ = x.shape
    xt = x.transpose(2, 0, 1).reshape(3, nb * npts)
    fa = _run_mlp(xt, W1, b1, W2, b2)
    partials = _run_sc_pool(fa.reshape(-1))
    merged = _run_merge(partials, nb)
    grid3 = merged.reshape(nb, OUT_DIM, CW, CH, CD)
    return jnp.pad(grid3, ((0, 0), (0, 0), (0, 1), (0, 1), (0, 1)))


# speedup vs baseline: 5.4749x; 1.0771x over previous
import dataclasses

import jax
import jax.numpy as jnp
from jax import lax
from jax.experimental import pallas as pl
from jax.experimental.pallas import tpu as pltpu
from jax.experimental.pallas import tpu_sc as plsc

C_DIM = 256
OUT_DIM = 16
CW, CH, CD = 16, 16, 16
NCELL = CW * CH * CD
PITCH = NCELL + 17
ACC_LEN = OUT_DIM * PITCH

NEG = -3.0e38


BLK = 4096


def _mlp_body(x_ref, w1t_ref, b1_ref, w2p_ref, rsp_ref, cb_ref, o_ref):
    xt = x_ref[...]
    h = lax.dot_general(xt.astype(jnp.bfloat16), w1t_ref[...],
                        (((0,), (0,)), ((), ())),
                        preferred_element_type=jnp.float32)
    hb = jnp.maximum(h + b1_ref[...], 0.0).astype(jnp.bfloat16)
    cell = jnp.clip(jnp.floor(xt * 16.0).astype(jnp.int32), 0, 15)
    hi = (cell[0:1, :] * CH + cell[1:2, :]).astype(jnp.bfloat16)
    lo = cell[2:3, :].astype(jnp.bfloat16)
    hl = jnp.concatenate([hi, lo], axis=0)
    sub = BLK // 4
    tot = cb_ref[...]
    for g in range(4):
        tot = tot + lax.dot_general(
            hb[g * sub:(g + 1) * sub], w2p_ref[g * C_DIM:(g + 1) * C_DIM],
            (((1,), (0,)), ((), ())), preferred_element_type=jnp.float32)
        tot = tot + lax.dot_general(
            hl[:, g * sub:(g + 1) * sub], rsp_ref[g * 2:g * 2 + 2],
            (((0,), (0,)), ((), ())), preferred_element_type=jnp.float32)
    ai = lax.bitcast_convert_type(tot.astype(jnp.int32), jnp.float32)
    is_feat = (lax.broadcasted_iota(jnp.int32, (sub, 128), 1)
               // OUT_DIM) % 2 == 0
    o_ref[...] = jnp.where(is_feat, tot, ai)


def _run_mlp(xt, W1, b1, W2, b2):
    m = xt.shape[1]
    grid = (m // BLK,)
    w2t = W2.T.astype(jnp.bfloat16)
    w2p = jnp.concatenate(
        [jnp.pad(w2t, ((0, 0), (g * 32, 112 - g * 32))) for g in range(4)],
        axis=0)
    rs = jnp.concatenate(
        [jnp.full((1, OUT_DIM), CD, jnp.bfloat16),
         jnp.full((1, OUT_DIM), 1, jnp.bfloat16)], axis=0)
    rsp = jnp.concatenate(
        [jnp.pad(rs, ((0, 0), (g * 32 + 16, 96 - g * 32))) for g in range(4)],
        axis=0)
    cb = jnp.tile(
        jnp.concatenate([b2, jnp.arange(OUT_DIM, dtype=jnp.float32) * PITCH]),
        4).reshape(1, 128)
    return pl.pallas_call(
        _mlp_body,
        grid=grid,
        in_specs=[
            pl.BlockSpec((3, BLK), lambda i: (0, i)),
            pl.BlockSpec((3, C_DIM), lambda i: (0, 0)),
            pl.BlockSpec((1, C_DIM), lambda i: (0, 0)),
            pl.BlockSpec((4 * C_DIM, 128), lambda i: (0, 0)),
            pl.BlockSpec((8, 128), lambda i: (0, 0)),
            pl.BlockSpec((1, 128), lambda i: (0, 0)),
        ],
        out_specs=pl.BlockSpec((BLK // 4, 128), lambda i: (i, 0)),
        out_shape=jax.ShapeDtypeStruct((m // 4, 128), jnp.float32),
    )(xt, W1.T.astype(jnp.bfloat16), b1.reshape(1, C_DIM), w2p, rsp, cb)


NWORK = 32
CHUNK = 512


CHW = CHUNK * 2 * OUT_DIM


def _sc_pool_body(fa_hbm, out_hbm, b0, b1, acc, s0, s1):
    c = lax.axis_index("c")
    s = lax.axis_index("s")
    wid = c * 16 + s
    npts = fa_hbm.shape[0] // (NWORK * 2 * OUT_DIM)
    nch = npts // CHUNK
    base = wid * npts * 2 * OUT_DIM

    @pl.loop(0, ACC_LEN, step=16)
    def _(i):
        acc[pl.ds(i, 16)] = jnp.full((16,), NEG, jnp.float32)

    pltpu.async_copy(fa_hbm.at[pl.ds(base, CHW)], b0, s0)
    pltpu.async_copy(fa_hbm.at[pl.ds(base + CHW, CHW)], b1, s1)

    def proc(buf):
        @pl.loop(0, CHUNK, step=4)
        def _(p):
            for u in range(4):
                val = buf[pl.ds((p + u) * 32, OUT_DIM)]
                idx = plsc.bitcast(
                    buf[pl.ds((p + u) * 32 + OUT_DIM, OUT_DIM)], jnp.int32)
                old = plsc.load_gather(acc, [idx])
                plsc.store_scatter(acc, [idx], jnp.maximum(old, val))

    @pl.loop(0, nch, step=2)
    def _(ci):
        r0 = base + ci * CHW
        pltpu.make_async_copy(fa_hbm.at[pl.ds(r0, CHW)], b0, s0).wait()
        proc(b0)

        @pl.when(ci + 2 < nch)
        def _():
            pltpu.async_copy(
                fa_hbm.at[pl.ds(base + (ci + 2) * CHW, CHW)], b0, s0)

        pltpu.make_async_copy(
            fa_hbm.at[pl.ds(base + (ci + 1) * CHW, CHW)], b1, s1).wait()
        proc(b1)

        @pl.when(ci + 3 < nch)
        def _():
            pltpu.async_copy(
                fa_hbm.at[pl.ds(base + (ci + 3) * CHW, CHW)], b1, s1)

    lane = lax.broadcasted_iota(jnp.int32, (16,), 0)

    @pl.loop(0, OUT_DIM)
    def _(f):
        @pl.loop(0, NCELL, step=16)
        def _(cc):
            b0[pl.ds(cc, 16)] = plsc.load_gather(
                acc, [f * PITCH + cc + lane])
        pltpu.sync_copy(b0.at[pl.ds(0, NCELL)],
                        out_hbm.at[wid, pl.ds(f * NCELL, NCELL)])


def _run_sc_pool(fa):
    mesh = plsc.VectorSubcoreMesh(core_axis_name="c", subcore_axis_name="s")
    cp = pltpu.CompilerParams()
    if "needs_layout_passes" in pltpu.CompilerParams.__dataclass_fields__:
        cp = dataclasses.replace(cp, needs_layout_passes=False)
    fn = pl.kernel(
        _sc_pool_body,
        out_type=jax.ShapeDtypeStruct((NWORK, OUT_DIM * NCELL), jnp.float32),
        mesh=mesh,
        scratch_types=[
            pltpu.VMEM((CHW,), jnp.float32),
            pltpu.VMEM((CHW,), jnp.float32),
            pltpu.VMEM((ACC_LEN,), jnp.float32),
            pltpu.SemaphoreType.DMA,
            pltpu.SemaphoreType.DMA,
        ],
        compiler_params=cp,
    )
    return fn(fa)


def _merge_body(p_ref, o_ref):
    m = jnp.maximum(p_ref[0, 0], p_ref[0, 1])
    o_ref[0] = jnp.where(m > -1.0e38, m, 0.0)


def _run_merge(partials, nb):
    p4 = partials.reshape(nb, 2, 512, 128)
    return pl.pallas_call(
        _merge_body,
        grid=(nb,),
        in_specs=[pl.BlockSpec((1, 2, 512, 128), lambda i: (i, 0, 0, 0))],
        out_specs=pl.BlockSpec((1, 512, 128), lambda i: (i, 0, 0)),
        out_shape=jax.ShapeDtypeStruct((nb, 512, 128), jnp.float32),
    )(p4)


def kernel(x, W1, b1, W2, b2):
    nb, npts, _ = x.shape
    xt = x.transpose(2, 0, 1).reshape(3, nb * npts)
    fa = _run_mlp(xt, W1, b1, W2, b2)
    partials = _run_sc_pool(fa.reshape(-1))
    merged = _run_merge(partials, nb)
    grid3 = merged.reshape(nb, OUT_DIM, CW, CH, CD)
    return jnp.pad(grid3, ((0, 0), (0, 0), (0, 1), (0, 1), (0, 1)))
